# baseline (speedup 1.0000x reference)
import jax
import jax.numpy as jnp
from jax.experimental import pallas as pl


def kernel(pixel_values, row_weight, col_weight):
    if pixel_values.ndim != 4:
        raise ValueError('pixel_values must be a 4D tensor')
    b, h, w, _ = pixel_values.shape
    dr = row_weight.shape[1]
    dc = col_weight.shape[1]
    d = dc + dr

    row_w = row_weight[:h]
    col_w = col_weight[:w]

    def body(col_ref, row_ref, out_ref):
        cw = col_ref[...]
        rw = row_ref[...]
        out_ref[0, :, :, :dc] = jnp.broadcast_to(cw[None, :, :], (h, w, dc))
        out_ref[0, :, :, dc:] = jnp.broadcast_to(rw[:, None, :], (h, w, dr))

    out = pl.pallas_call(
        body,
        grid=(b,),
        in_specs=[
            pl.BlockSpec((w, dc), lambda i: (0, 0)),
            pl.BlockSpec((h, dr), lambda i: (0, 0)),
        ],
        out_specs=pl.BlockSpec((1, h, w, d), lambda i: (i, 0, 0, 0)),
        out_shape=jax.ShapeDtypeStruct((b, h, w, d), jnp.float32),
    )(col_w, row_w)
    return out
